# Initial kernel scaffold; baseline (speedup 1.0000x reference)
import jax
import jax.numpy as jnp
from jax.experimental import pallas as pl


def kernel(x, edge_index, batch, w1_0, b1_0, g_0, be_0, rm_0, rv_0, w2_0, b2_0, w1_1, b1_1, g_1, be_1, rm_1, rv_1, w2_1, b2_1, w1_2, b1_2, g_2, be_2, rm_2, rv_2, w2_2, b2_2, gf, bef, rmf, rvf, wfc, bfc):
    raise NotImplementedError("write your pallas kernel here")


# speedup vs baseline: 4.5689x; 4.5689x over previous
import functools

import jax
import jax.numpy as jnp
from jax import lax
from jax.experimental import pallas as pl
from jax.experimental.pallas import tpu as pltpu
from jax.experimental.pallas import tpu_sc as plsc

N = 10000
E = 320000
D = 128
D_LAT = 64
G = 64

NC = 2
NS = 16
NW = NC * NS
EPW = E // NW
CHUNK = 80
NCHUNK = EPW // CHUNK
NPAD = 10240
RPT = NPAD // NS

BLK = 1000
NBLK = N // BLK


def _sc_aggregate(h, src, dst, zrows):
    mesh = plsc.VectorSubcoreMesh(core_axis_name="c", subcore_axis_name="s")

    @functools.partial(
        pl.kernel,
        mesh=mesh,
        out_type=jax.ShapeDtypeStruct((NC, NPAD, D), jnp.float32),
        scratch_types=[
            pltpu.VMEM((CHUNK,), jnp.int32),
            pltpu.VMEM((CHUNK,), jnp.int32),
            pltpu.VMEM((CHUNK, D), jnp.float32),
            pltpu.VMEM_SHARED((NPAD, D), jnp.float32),
            pltpu.SemaphoreType.DMA,
        ],
    )
    def agg_kernel(h_hbm, src_hbm, dst_hbm, z_hbm, out_hbm, sidx, didx, rows,
                   acc, sem):
        c = lax.axis_index("c")
        s = lax.axis_index("s")
        wid = s * NC + c

        pltpu.sync_copy(z_hbm, acc.at[pl.ds(s * RPT, RPT)])
        plsc.subcore_barrier()

        def body(i, carry):
            base = wid * EPW + i * CHUNK
            pltpu.sync_copy(src_hbm.at[pl.ds(base, CHUNK)], sidx)
            pltpu.sync_copy(dst_hbm.at[pl.ds(base, CHUNK)], didx)
            pltpu.async_copy(h_hbm.at[sidx], rows, sem).wait()
            pltpu.sync_copy(rows, acc.at[didx], add=True)
            return carry

        lax.fori_loop(0, NCHUNK, body, 0)
        plsc.subcore_barrier()
        pltpu.sync_copy(acc.at[pl.ds(s * RPT, RPT)],
                        out_hbm.at[c, pl.ds(s * RPT, RPT)])

    return agg_kernel(h, src, dst, zrows)


def _leaky(v):
    return jnp.where(v > 0, v, 0.2 * v)


def _mlp_body(h_ref, parts_ref, w1_ref, b1_ref, w2_ref, b2_ref, out_ref):
    hb = h_ref[...] + parts_ref[0] + parts_ref[1]
    t = _leaky(jnp.dot(hb, w1_ref[...], preferred_element_type=jnp.float32)
               + b1_ref[...])
    t = _leaky(jnp.dot(t, w2_ref[...], preferred_element_type=jnp.float32)
               + b2_ref[...])
    out_ref[...] = t


def _mlp(h, parts, w1, b1, w2p, b2p):
    return pl.pallas_call(
        _mlp_body,
        grid=(NBLK,),
        in_specs=[
            pl.BlockSpec((BLK, D), lambda i: (i, 0)),
            pl.BlockSpec((NC, BLK, D), lambda i: (0, i, 0)),
            pl.BlockSpec((D, D), lambda i: (0, 0)),
            pl.BlockSpec((1, D), lambda i: (0, 0)),
            pl.BlockSpec((D, D), lambda i: (0, 0)),
            pl.BlockSpec((1, D), lambda i: (0, 0)),
        ],
        out_specs=pl.BlockSpec((BLK, D), lambda i: (i, 0)),
        out_shape=jax.ShapeDtypeStruct((N, D), jnp.float32),
    )(h, parts, w1, b1, w2p, b2p)


def _final_body(h_ref, parts_ref, batch_ref, w1_ref, b1_ref, w2_ref, b2_ref,
                wf_ref, bf_ref, out_ref, acc_ref):
    i = pl.program_id(0)
    hb = h_ref[...] + parts_ref[0] + parts_ref[1]
    t = _leaky(jnp.dot(hb, w1_ref[...], preferred_element_type=jnp.float32)
               + b1_ref[...])
    t = _leaky(jnp.dot(t, w2_ref[...], preferred_element_type=jnp.float32)
               + b2_ref[...])
    seg = batch_ref[0, 0].reshape(BLK, 1)
    onehot = (seg == lax.broadcasted_iota(jnp.int32, (BLK, G), 1)
              ).astype(jnp.float32)
    p = lax.dot_general(onehot, t, (((0,), (0,)), ((), ())),
                        preferred_element_type=jnp.float32)

    @pl.when(i == 0)
    def _():
        acc_ref[...] = p

    @pl.when(i > 0)
    def _():
        acc_ref[...] += p

    @pl.when(i == NBLK - 1)
    def _():
        out_ref[...] = (jnp.dot(acc_ref[...], wf_ref[...],
                                preferred_element_type=jnp.float32)
                        + bf_ref[...])


def _final(h, parts, batch3, w1, b1, w2p, b2p, wfp, bfp):
    return pl.pallas_call(
        _final_body,
        grid=(NBLK,),
        in_specs=[
            pl.BlockSpec((BLK, D), lambda i: (i, 0)),
            pl.BlockSpec((NC, BLK, D), lambda i: (0, i, 0)),
            pl.BlockSpec((1, 1, BLK), lambda i: (i, 0, 0)),
            pl.BlockSpec((D, D), lambda i: (0, 0)),
            pl.BlockSpec((1, D), lambda i: (0, 0)),
            pl.BlockSpec((D, D), lambda i: (0, 0)),
            pl.BlockSpec((1, D), lambda i: (0, 0)),
            pl.BlockSpec((D, D_LAT), lambda i: (0, 0)),
            pl.BlockSpec((1, D_LAT), lambda i: (0, 0)),
        ],
        out_specs=pl.BlockSpec((G, D_LAT), lambda i: (0, 0)),
        out_shape=jax.ShapeDtypeStruct((G, D_LAT), jnp.float32),
        scratch_shapes=[pltpu.VMEM((G, D), jnp.float32)],
    )(h, parts, batch3, w1, b1, w2p, b2p, wfp, bfp)


def _fold_bn(g, be, rm, rv, w2, b2):
    scale = g / jnp.sqrt(rv + 1e-5)
    shift = be - rm * scale
    return scale[:, None] * w2, b2 + shift @ w2


def kernel(x, edge_index, batch, w1_0, b1_0, g_0, be_0, rm_0, rv_0, w2_0,
           b2_0, w1_1, b1_1, g_1, be_1, rm_1, rv_1, w2_1, b2_1, w1_2, b1_2,
           g_2, be_2, rm_2, rv_2, w2_2, b2_2, gf, bef, rmf, rvf, wfc, bfc):
    src = edge_index[0]
    dst = edge_index[1]
    zrows = jnp.zeros((RPT, D), jnp.float32)
    batch3 = batch.reshape(NBLK, 1, BLK)

    w2p0, b2p0 = _fold_bn(g_0, be_0, rm_0, rv_0, w2_0, b2_0)
    w2p1, b2p1 = _fold_bn(g_1, be_1, rm_1, rv_1, w2_1, b2_1)
    w2p2, b2p2 = _fold_bn(g_2, be_2, rm_2, rv_2, w2_2, b2_2)
    scale_f = gf / jnp.sqrt(rvf + 1e-5)
    shift_f = bef - rmf * scale_f
    wfp = scale_f[:, None] * wfc
    bfp = bfc + shift_f @ wfc

    parts = _sc_aggregate(x, src, dst, zrows)
    h = _mlp(x, parts, w1_0, b1_0.reshape(1, D), w2p0, b2p0.reshape(1, D))
    parts = _sc_aggregate(h, src, dst, zrows)
    h = _mlp(h, parts, w1_1, b1_1.reshape(1, D), w2p1, b2p1.reshape(1, D))
    parts = _sc_aggregate(h, src, dst, zrows)
    out = _final(h, parts, batch3, w1_2, b1_2.reshape(1, D), w2p2,
                 b2p2.reshape(1, D), wfp, bfp.reshape(1, D_LAT))
    return out


# speedup vs baseline: 9.9436x; 2.1764x over previous
import functools

import jax
import jax.numpy as jnp
from jax import lax
from jax.experimental import pallas as pl
from jax.experimental.pallas import tpu as pltpu
from jax.experimental.pallas import tpu_sc as plsc

N = 10000
E = 320000
D = 128
D_LAT = 64
G = 64

NC = 2
NS = 16
NW = NC * NS
EPW = E // NW
CHUNK = 80
NCHUNK = EPW // CHUNK
IBLK = 25
NBLKI = NCHUNK // IBLK
NPAD = 10240
RPT = NPAD // NS

BLK = 1000
NBLK = N // BLK


def _sc_aggregate(h, src3, dst3, zrows):
    mesh = plsc.VectorSubcoreMesh(core_axis_name="c", subcore_axis_name="s")

    @functools.partial(
        pl.kernel,
        mesh=mesh,
        out_type=jax.ShapeDtypeStruct((NC, NPAD, D), jnp.float32),
        scratch_types=[
            pltpu.VMEM((IBLK, CHUNK), jnp.int32),
            pltpu.VMEM((IBLK, CHUNK), jnp.int32),
            pltpu.VMEM((CHUNK, D), jnp.float32),
            pltpu.VMEM((CHUNK, D), jnp.float32),
            pltpu.VMEM_SHARED((NPAD, D), jnp.float32),
            pltpu.SemaphoreType.DMA,
            pltpu.SemaphoreType.DMA,
        ],
    )
    def agg_kernel(h_hbm, src_hbm, dst_hbm, z_hbm, out_hbm, srcv, dstv,
                   rows0, rows1, acc, sem0, sem1):
        c = lax.axis_index("c")
        s = lax.axis_index("s")
        wid = s * NC + c
        rows = (rows0, rows1)
        sems = (sem0, sem1)

        pltpu.sync_copy(z_hbm, acc.at[pl.ds(s * RPT, RPT)])
        plsc.subcore_barrier()

        def block(j, carry):
            pltpu.sync_copy(src_hbm.at[wid, j], srcv)
            pltpu.sync_copy(dst_hbm.at[wid, j], dstv)
            pltpu.async_copy(h_hbm.at[srcv.at[0]], rows0, sem0)
            pltpu.async_copy(h_hbm.at[srcv.at[1]], rows1, sem1)

            def pair(i0, cr):
                for b in range(2):
                    i = i0 + b
                    pltpu.make_async_copy(h_hbm.at[srcv.at[i]], rows[b],
                                          sems[b]).wait()
                    pltpu.sync_copy(rows[b], acc.at[dstv.at[i]], add=True)

                    @pl.when(i + 2 < IBLK)
                    def _():
                        pltpu.async_copy(h_hbm.at[srcv.at[i + 2]], rows[b],
                                         sems[b])
                return cr

            lax.fori_loop(0, IBLK // 2, lambda k, cr: pair(k * 2, cr), 0)
            pltpu.make_async_copy(h_hbm.at[srcv.at[IBLK - 1]], rows0,
                                  sem0).wait()
            pltpu.sync_copy(rows0, acc.at[dstv.at[IBLK - 1]], add=True)
            return carry

        lax.fori_loop(0, NBLKI, block, 0)
        plsc.subcore_barrier()
        pltpu.sync_copy(acc.at[pl.ds(s * RPT, RPT)],
                        out_hbm.at[c, pl.ds(s * RPT, RPT)])

    return agg_kernel(h, src3, dst3, zrows)


def _leaky(v):
    return jnp.where(v > 0, v, 0.2 * v)


def _mlp_body(h_ref, parts_ref, w1_ref, b1_ref, w2_ref, b2_ref, out_ref):
    hb = h_ref[...] + parts_ref[0] + parts_ref[1]
    t = _leaky(jnp.dot(hb, w1_ref[...], preferred_element_type=jnp.float32)
               + b1_ref[...])
    t = _leaky(jnp.dot(t, w2_ref[...], preferred_element_type=jnp.float32)
               + b2_ref[...])
    out_ref[...] = t


def _mlp(h, parts, w1, b1, w2p, b2p):
    return pl.pallas_call(
        _mlp_body,
        grid=(NBLK,),
        in_specs=[
            pl.BlockSpec((BLK, D), lambda i: (i, 0)),
            pl.BlockSpec((NC, BLK, D), lambda i: (0, i, 0)),
            pl.BlockSpec((D, D), lambda i: (0, 0)),
            pl.BlockSpec((1, D), lambda i: (0, 0)),
            pl.BlockSpec((D, D), lambda i: (0, 0)),
            pl.BlockSpec((1, D), lambda i: (0, 0)),
        ],
        out_specs=pl.BlockSpec((BLK, D), lambda i: (i, 0)),
        out_shape=jax.ShapeDtypeStruct((N, D), jnp.float32),
    )(h, parts, w1, b1, w2p, b2p)


def _final_body(h_ref, parts_ref, batch_ref, w1_ref, b1_ref, w2_ref, b2_ref,
                wf_ref, bf_ref, out_ref, acc_ref):
    i = pl.program_id(0)
    hb = h_ref[...] + parts_ref[0] + parts_ref[1]
    t = _leaky(jnp.dot(hb, w1_ref[...], preferred_element_type=jnp.float32)
               + b1_ref[...])
    t = _leaky(jnp.dot(t, w2_ref[...], preferred_element_type=jnp.float32)
               + b2_ref[...])
    seg = batch_ref[0, 0].reshape(BLK, 1)
    onehot = (seg == lax.broadcasted_iota(jnp.int32, (BLK, G), 1)
              ).astype(jnp.float32)
    p = lax.dot_general(onehot, t, (((0,), (0,)), ((), ())),
                        preferred_element_type=jnp.float32)

    @pl.when(i == 0)
    def _():
        acc_ref[...] = p

    @pl.when(i > 0)
    def _():
        acc_ref[...] += p

    @pl.when(i == NBLK - 1)
    def _():
        out_ref[...] = (jnp.dot(acc_ref[...], wf_ref[...],
                                preferred_element_type=jnp.float32)
                        + bf_ref[...])


def _final(h, parts, batch3, w1, b1, w2p, b2p, wfp, bfp):
    return pl.pallas_call(
        _final_body,
        grid=(NBLK,),
        in_specs=[
            pl.BlockSpec((BLK, D), lambda i: (i, 0)),
            pl.BlockSpec((NC, BLK, D), lambda i: (0, i, 0)),
            pl.BlockSpec((1, 1, BLK), lambda i: (i, 0, 0)),
            pl.BlockSpec((D, D), lambda i: (0, 0)),
            pl.BlockSpec((1, D), lambda i: (0, 0)),
            pl.BlockSpec((D, D), lambda i: (0, 0)),
            pl.BlockSpec((1, D), lambda i: (0, 0)),
            pl.BlockSpec((D, D_LAT), lambda i: (0, 0)),
            pl.BlockSpec((1, D_LAT), lambda i: (0, 0)),
        ],
        out_specs=pl.BlockSpec((G, D_LAT), lambda i: (0, 0)),
        out_shape=jax.ShapeDtypeStruct((G, D_LAT), jnp.float32),
        scratch_shapes=[pltpu.VMEM((G, D), jnp.float32)],
    )(h, parts, batch3, w1, b1, w2p, b2p, wfp, bfp)


def _fold_bn(g, be, rm, rv, w2, b2):
    scale = g / jnp.sqrt(rv + 1e-5)
    shift = be - rm * scale
    return scale[:, None] * w2, b2 + shift @ w2


def kernel(x, edge_index, batch, w1_0, b1_0, g_0, be_0, rm_0, rv_0, w2_0,
           b2_0, w1_1, b1_1, g_1, be_1, rm_1, rv_1, w2_1, b2_1, w1_2, b1_2,
           g_2, be_2, rm_2, rv_2, w2_2, b2_2, gf, bef, rmf, rvf, wfc, bfc):
    src = edge_index[0].reshape(NW, NBLKI, IBLK, CHUNK)
    dst = edge_index[1].reshape(NW, NBLKI, IBLK, CHUNK)
    zrows = jnp.zeros((RPT, D), jnp.float32)
    batch3 = batch.reshape(NBLK, 1, BLK)

    w2p0, b2p0 = _fold_bn(g_0, be_0, rm_0, rv_0, w2_0, b2_0)
    w2p1, b2p1 = _fold_bn(g_1, be_1, rm_1, rv_1, w2_1, b2_1)
    w2p2, b2p2 = _fold_bn(g_2, be_2, rm_2, rv_2, w2_2, b2_2)
    scale_f = gf / jnp.sqrt(rvf + 1e-5)
    shift_f = bef - rmf * scale_f
    wfp = scale_f[:, None] * wfc
    bfp = bfc + shift_f @ wfc

    parts = _sc_aggregate(x, src, dst, zrows)
    h = _mlp(x, parts, w1_0, b1_0.reshape(1, D), w2p0, b2p0.reshape(1, D))
    parts = _sc_aggregate(h, src, dst, zrows)
    h = _mlp(h, parts, w1_1, b1_1.reshape(1, D), w2p1, b2p1.reshape(1, D))
    parts = _sc_aggregate(h, src, dst, zrows)
    out = _final(h, parts, batch3, w1_2, b1_2.reshape(1, D), w2p2,
                 b2p2.reshape(1, D), wfp, bfp.reshape(1, D_LAT))
    return out


# speedup vs baseline: 11.3568x; 1.1421x over previous
import functools

import jax
import jax.numpy as jnp
from jax import lax
from jax.experimental import pallas as pl
from jax.experimental.pallas import tpu as pltpu
from jax.experimental.pallas import tpu_sc as plsc

N = 10000
E = 320000
D = 128
D_LAT = 64
G = 64

NC = 2
NS = 16
NW = NC * NS
EPW = E // NW
CHUNK = 40
NCHUNK = EPW // CHUNK
IBLK = 50
NBLKI = NCHUNK // IBLK
NSLOT = 5
LOOK = 3
NPAD = 10112
RPT = NPAD // NS

BLK = 1000
NBLK = N // BLK


def _sc_aggregate(h, src3, dst3, zrows):
    mesh = plsc.VectorSubcoreMesh(core_axis_name="c", subcore_axis_name="s")

    @functools.partial(
        pl.kernel,
        mesh=mesh,
        out_type=jax.ShapeDtypeStruct((NC, NPAD, D), jnp.float32),
        scratch_types=(
            [pltpu.VMEM((IBLK, CHUNK), jnp.int32)] * 2
            + [pltpu.VMEM((CHUNK, D), jnp.float32)] * NSLOT
            + [pltpu.VMEM_SHARED((NPAD, D), jnp.float32)]
            + [pltpu.SemaphoreType.DMA] * (2 * NSLOT)
        ),
    )
    def agg_kernel(h_hbm, src_hbm, dst_hbm, z_hbm, out_hbm, srcv, dstv,
                   *rest):
        rows = rest[:NSLOT]
        acc = rest[NSLOT]
        gsem = rest[NSLOT + 1:2 * NSLOT + 1]
        ssem = rest[2 * NSLOT + 1:]
        c = lax.axis_index("c")
        s = lax.axis_index("s")
        wid = s * NC + c

        pltpu.sync_copy(z_hbm, acc.at[pl.ds(s * RPT, RPT)])
        plsc.subcore_barrier()

        def block(j, carry):
            pltpu.sync_copy(src_hbm.at[wid, j], srcv)
            pltpu.sync_copy(dst_hbm.at[wid, j], dstv)
            for b in range(LOOK):
                pltpu.async_copy(h_hbm.at[srcv.at[b]], rows[b], gsem[b])

            def group(k, cr):
                for b in range(NSLOT):
                    i = k * NSLOT + b
                    ip = i + LOOK
                    bb = (b + LOOK) % NSLOT

                    @pl.when(ip < IBLK)
                    def _():
                        @pl.when(ip >= NSLOT)
                        def _():
                            pltpu.make_async_copy(
                                rows[bb], acc.at[dstv.at[ip - NSLOT]],
                                ssem[bb]).wait()

                        pltpu.async_copy(h_hbm.at[srcv.at[ip]], rows[bb],
                                         gsem[bb])

                    pltpu.make_async_copy(h_hbm.at[srcv.at[i]], rows[b],
                                          gsem[b]).wait()
                    pltpu.async_copy(rows[b], acc.at[dstv.at[i]], ssem[b],
                                     add=True)
                return cr

            lax.fori_loop(0, IBLK // NSLOT, group, 0)
            for b in range(NSLOT):
                il = IBLK - NSLOT + b
                pltpu.make_async_copy(rows[b], acc.at[dstv.at[il]],
                                      ssem[b]).wait()
            return carry

        lax.fori_loop(0, NBLKI, block, 0)
        plsc.subcore_barrier()
        pltpu.sync_copy(acc.at[pl.ds(s * RPT, RPT)],
                        out_hbm.at[c, pl.ds(s * RPT, RPT)])

    return agg_kernel(h, src3, dst3, zrows)


def _leaky(v):
    return jnp.where(v > 0, v, 0.2 * v)


def _mlp_body(h_ref, parts_ref, w1_ref, b1_ref, w2_ref, b2_ref, out_ref):
    hb = h_ref[...] + parts_ref[0] + parts_ref[1]
    t = _leaky(jnp.dot(hb, w1_ref[...], preferred_element_type=jnp.float32)
               + b1_ref[...])
    t = _leaky(jnp.dot(t, w2_ref[...], preferred_element_type=jnp.float32)
               + b2_ref[...])
    out_ref[...] = t


def _mlp(h, parts, w1, b1, w2p, b2p):
    return pl.pallas_call(
        _mlp_body,
        grid=(NBLK,),
        in_specs=[
            pl.BlockSpec((BLK, D), lambda i: (i, 0)),
            pl.BlockSpec((NC, BLK, D), lambda i: (0, i, 0)),
            pl.BlockSpec((D, D), lambda i: (0, 0)),
            pl.BlockSpec((1, D), lambda i: (0, 0)),
            pl.BlockSpec((D, D), lambda i: (0, 0)),
            pl.BlockSpec((1, D), lambda i: (0, 0)),
        ],
        out_specs=pl.BlockSpec((BLK, D), lambda i: (i, 0)),
        out_shape=jax.ShapeDtypeStruct((N, D), jnp.float32),
    )(h, parts, w1, b1, w2p, b2p)


def _final_body(h_ref, parts_ref, batch_ref, w1_ref, b1_ref, w2_ref, b2_ref,
                wf_ref, bf_ref, out_ref, acc_ref):
    i = pl.program_id(0)
    hb = h_ref[...] + parts_ref[0] + parts_ref[1]
    t = _leaky(jnp.dot(hb, w1_ref[...], preferred_element_type=jnp.float32)
               + b1_ref[...])
    t = _leaky(jnp.dot(t, w2_ref[...], preferred_element_type=jnp.float32)
               + b2_ref[...])
    seg = batch_ref[0, 0].reshape(BLK, 1)
    onehot = (seg == lax.broadcasted_iota(jnp.int32, (BLK, G), 1)
              ).astype(jnp.float32)
    p = lax.dot_general(onehot, t, (((0,), (0,)), ((), ())),
                        preferred_element_type=jnp.float32)

    @pl.when(i == 0)
    def _():
        acc_ref[...] = p

    @pl.when(i > 0)
    def _():
        acc_ref[...] += p

    @pl.when(i == NBLK - 1)
    def _():
        out_ref[...] = (jnp.dot(acc_ref[...], wf_ref[...],
                                preferred_element_type=jnp.float32)
                        + bf_ref[...])


def _final(h, parts, batch3, w1, b1, w2p, b2p, wfp, bfp):
    return pl.pallas_call(
        _final_body,
        grid=(NBLK,),
        in_specs=[
            pl.BlockSpec((BLK, D), lambda i: (i, 0)),
            pl.BlockSpec((NC, BLK, D), lambda i: (0, i, 0)),
            pl.BlockSpec((1, 1, BLK), lambda i: (i, 0, 0)),
            pl.BlockSpec((D, D), lambda i: (0, 0)),
            pl.BlockSpec((1, D), lambda i: (0, 0)),
            pl.BlockSpec((D, D), lambda i: (0, 0)),
            pl.BlockSpec((1, D), lambda i: (0, 0)),
            pl.BlockSpec((D, D_LAT), lambda i: (0, 0)),
            pl.BlockSpec((1, D_LAT), lambda i: (0, 0)),
        ],
        out_specs=pl.BlockSpec((G, D_LAT), lambda i: (0, 0)),
        out_shape=jax.ShapeDtypeStruct((G, D_LAT), jnp.float32),
        scratch_shapes=[pltpu.VMEM((G, D), jnp.float32)],
    )(h, parts, batch3, w1, b1, w2p, b2p, wfp, bfp)


def _fold_bn(g, be, rm, rv, w2, b2):
    scale = g / jnp.sqrt(rv + 1e-5)
    shift = be - rm * scale
    return scale[:, None] * w2, b2 + shift @ w2


def kernel(x, edge_index, batch, w1_0, b1_0, g_0, be_0, rm_0, rv_0, w2_0,
           b2_0, w1_1, b1_1, g_1, be_1, rm_1, rv_1, w2_1, b2_1, w1_2, b1_2,
           g_2, be_2, rm_2, rv_2, w2_2, b2_2, gf, bef, rmf, rvf, wfc, bfc):
    src = edge_index[0].reshape(NW, NBLKI, IBLK, CHUNK)
    dst = edge_index[1].reshape(NW, NBLKI, IBLK, CHUNK)
    zrows = jnp.zeros((RPT, D), jnp.float32)
    batch3 = batch.reshape(NBLK, 1, BLK)

    w2p0, b2p0 = _fold_bn(g_0, be_0, rm_0, rv_0, w2_0, b2_0)
    w2p1, b2p1 = _fold_bn(g_1, be_1, rm_1, rv_1, w2_1, b2_1)
    w2p2, b2p2 = _fold_bn(g_2, be_2, rm_2, rv_2, w2_2, b2_2)
    scale_f = gf / jnp.sqrt(rvf + 1e-5)
    shift_f = bef - rmf * scale_f
    wfp = scale_f[:, None] * wfc
    bfp = bfc + shift_f @ wfc

    parts = _sc_aggregate(x, src, dst, zrows)
    h = _mlp(x, parts, w1_0, b1_0.reshape(1, D), w2p0, b2p0.reshape(1, D))
    parts = _sc_aggregate(h, src, dst, zrows)
    h = _mlp(h, parts, w1_1, b1_1.reshape(1, D), w2p1, b2p1.reshape(1, D))
    parts = _sc_aggregate(h, src, dst, zrows)
    out = _final(h, parts, batch3, w1_2, b1_2.reshape(1, D), w2p2,
                 b2p2.reshape(1, D), wfp, bfp.reshape(1, D_LAT))
    return out
